# Initial kernel scaffold; baseline (speedup 1.0000x reference)
import jax
import jax.numpy as jnp
from jax.experimental import pallas as pl


def kernel(x, edge_index, W1, b1, g1, be1, W2, b2, g2, be2, W3, b3, g3, be3, Wl1, bl1, Wl2, bl2, g4, be4, Wo, bo):
    raise NotImplementedError("write your pallas kernel here")


# speedup vs baseline: 4.8797x; 4.8797x over previous
import functools

import jax
import jax.numpy as jnp
from jax import lax
from jax.experimental import pallas as pl
from jax.experimental.pallas import tpu as pltpu
from jax.experimental.pallas import tpu_sc as plsc

NW = 32
NC, NS, L = 2, 16, 16
G = 128
CAPV = 32768
LRELU = 0.01


def _mesh():
    return plsc.VectorSubcoreMesh(
        core_axis_name="c", subcore_axis_name="s", num_cores=NC,
        num_subcores=NS)


def _wid():
    return lax.axis_index("s") * NC + lax.axis_index("c")


def _make_prep(E, R, EB):
    NPAD = NW * R

    @functools.partial(
        pl.kernel,
        out_type=[
            jax.ShapeDtypeStruct((NW, CAPV), jnp.int32),
            jax.ShapeDtypeStruct((NW, CAPV), jnp.int32),
            jax.ShapeDtypeStruct((NW, 16), jnp.int32),
            jax.ShapeDtypeStruct((NPAD, 16), jnp.float32),
        ],
        mesh=_mesh(),
        scratch_types=[
            pltpu.VMEM((EB,), jnp.int32),
            pltpu.VMEM((EB,), jnp.int32),
            pltpu.VMEM((CAPV + 16,), jnp.int32),
            pltpu.VMEM((CAPV + 16,), jnp.int32),
            pltpu.VMEM((R, 16), jnp.float32),
            pltpu.VMEM((16,), jnp.int32),
        ],
        compiler_params=pltpu.CompilerParams(needs_layout_passes=False),
    )
    def prep(src_hbm, dst_hbm, src_out, dstl_out, cnt_out, deg_out,
             src_b, dst_b, cs_v, cd_v, deg_v, cnt_v):
        wid = _wid()
        lo = wid * R
        zero16 = jnp.zeros((16,), jnp.float32)

        def dinit(i, _):
            deg_v[i, pl.ds(0, 16)] = zero16
            return 0
        lax.fori_loop(0, R, dinit, 0)

        def blk(b, cnt):
            pltpu.sync_copy(src_hbm.at[pl.ds(b * EB, EB)], src_b)
            pltpu.sync_copy(dst_hbm.at[pl.ds(b * EB, EB)], dst_b)

            def vec(i, cnt):
                d = dst_b[pl.ds(i * 16, 16)]
                s = src_b[pl.ds(i * 16, 16)]
                m = (d >= lo) & (d < lo + R)
                pos = plsc.cumsum(m.astype(jnp.int32))
                idx = jnp.minimum(cnt + pos - 1, CAPV - G - 1)
                plsc.store_scatter(cs_v, [idx], s, mask=m)
                plsc.store_scatter(cd_v, [idx], d - lo, mask=m)
                return cnt + jnp.sum(m.astype(jnp.int32))
            return lax.fori_loop(0, EB // 16, vec, cnt)

        cnt = lax.fori_loop(0, E // EB, blk, jnp.int32(0))
        cnt = jnp.minimum(cnt, CAPV - G)

        iot = lax.iota(jnp.int32, 16)
        dump = jnp.full((16,), R + 4, jnp.int32)
        for k in range(G // 16):
            idx = cnt + iot + k * 16
            spread = (wid * 331 + idx * 997) % (NPAD - R)
            plsc.store_scatter(cs_v, [idx], spread, mask=None)
            plsc.store_scatter(cd_v, [idx], dump, mask=None)

        onehot = jnp.where(iot == 0, 1.0, 0.0).astype(jnp.float32)

        def dloop(j, _):
            dl = cd_v[pl.ds(j, 16)][0]
            plsc.addupdate(deg_v.at[dl, pl.ds(0, 16)], onehot)
            return 0
        lax.fori_loop(0, cnt, dloop, 0)

        cnt_v[...] = jnp.full((16,), cnt, jnp.int32)
        pltpu.sync_copy(cs_v.at[pl.ds(0, CAPV)], src_out.at[wid])
        pltpu.sync_copy(cd_v.at[pl.ds(0, CAPV)], dstl_out.at[wid])
        pltpu.sync_copy(cnt_v, cnt_out.at[wid])
        pltpu.sync_copy(deg_v, deg_out.at[pl.ds(lo, R)])

    return prep


def _make_scatter(R, C):
    NPAD = NW * R
    ACC_R = R + 8

    @functools.partial(
        pl.kernel,
        out_type=[
            jax.ShapeDtypeStruct((NPAD, C), jnp.float32),
            jax.ShapeDtypeStruct((NPAD, C), jnp.float32),
        ],
        mesh=_mesh(),
        scratch_types=[
            pltpu.VMEM((G,), jnp.int32),
            pltpu.VMEM((G + 16,), jnp.int32),
            pltpu.VMEM((G, C), jnp.float32),
            pltpu.VMEM((ACC_R, C), jnp.float32),
            pltpu.VMEM((ACC_R, C), jnp.float32),
            pltpu.VMEM((16,), jnp.int32),
            pltpu.SemaphoreType.DMA,
        ],
        compiler_params=pltpu.CompilerParams(
            needs_layout_passes=False, use_tc_tiling_on_sc=False),
    )
    def scat(b_hbm, src_hbm, dstl_hbm, cnt_hbm, sb_out, mb_out,
             sidx, dlv, buf, accs, accm, cntv, sem):
        wid = _wid()
        pltpu.sync_copy(cnt_hbm.at[wid], cntv)
        cnt = cntv[...][0]
        nch = (cnt + G - 1) // G

        zero16 = jnp.zeros((16,), jnp.float32)
        neg16 = jnp.full((16,), -1e30, jnp.float32)

        def init(i, _):
            for k in range(C // 16):
                accs[i, pl.ds(k * 16, 16)] = zero16
                accm[i, pl.ds(k * 16, 16)] = neg16
            return 0
        lax.fori_loop(0, ACC_R, init, 0)

        def chunk(g, _):
            pltpu.sync_copy(src_hbm.at[wid, pl.ds(g * G, G)], sidx)
            pltpu.sync_copy(dstl_hbm.at[wid, pl.ds(g * G, G)],
                            dlv.at[pl.ds(0, G)])
            pltpu.async_copy(b_hbm.at[sidx], buf, sem).wait()

            def edge(j, _):
                dl = dlv[pl.ds(j, 16)][0]
                for k in range(C // 16):
                    v = buf[j, pl.ds(k * 16, 16)]
                    mx = accm[dl, pl.ds(k * 16, 16)]
                    accm[dl, pl.ds(k * 16, 16)] = jnp.maximum(mx, v)
                    plsc.addupdate(accs.at[dl, pl.ds(k * 16, 16)], v)
                return 0
            lax.fori_loop(0, G, edge, 0)
            return 0
        lax.fori_loop(0, nch, chunk, 0)

        pltpu.sync_copy(accs.at[pl.ds(0, R)], sb_out.at[pl.ds(wid * R, R)])
        pltpu.sync_copy(accm.at[pl.ds(0, R)], mb_out.at[pl.ds(wid * R, R)])

    return scat


def _grid_rows(NPAD):
    BR = 1024 if NPAD % 1024 == 0 else NPAD
    return BR, NPAD // BR


def _tc_pre(x, wt, wb, b):
    NPAD, F = x.shape
    C = wt.shape[1]
    BR, NB = _grid_rows(NPAD)

    def body(x_ref, wt_ref, wb_ref, b_ref, a_out, b_out):
        xv = x_ref[...]
        a_out[...] = jnp.dot(xv, wt_ref[...],
                             preferred_element_type=jnp.float32) + b_ref[...]
        b_out[...] = jnp.dot(xv, wb_ref[...],
                             preferred_element_type=jnp.float32)

    return pl.pallas_call(
        body,
        grid=(NB,),
        in_specs=[
            pl.BlockSpec((BR, F), lambda i: (i, 0)),
            pl.BlockSpec((F, C), lambda i: (0, 0)),
            pl.BlockSpec((F, C), lambda i: (0, 0)),
            pl.BlockSpec((1, C), lambda i: (0, 0)),
        ],
        out_specs=[
            pl.BlockSpec((BR, C), lambda i: (i, 0)),
            pl.BlockSpec((BR, C), lambda i: (i, 0)),
        ],
        out_shape=[
            jax.ShapeDtypeStruct((NPAD, C), jnp.float32),
            jax.ShapeDtypeStruct((NPAD, C), jnp.float32),
        ],
    )(x, wt, wb, b.reshape(1, C))


def _combine(A, SB, MB, dg):
    s = dg * A + SB
    mean = s / jnp.maximum(dg, 1.0)
    mx = jnp.where(dg > 0.0, A + MB, 0.0)
    return jnp.concatenate([mx, mean, s], axis=1)


def _tc_stats(N, A, SB, MB, deg):
    NPAD, C = A.shape
    BR, NB = _grid_rows(NPAD)

    def body(a_ref, sb_ref, mb_ref, dg_ref, s1_out, s2_out):
        i = pl.program_id(0)
        dg = dg_ref[...][:, 0:1]
        h = _combine(a_ref[...], sb_ref[...], mb_ref[...], dg)
        rows = lax.broadcasted_iota(jnp.int32, (BR, 1), 0) + i * BR
        h = jnp.where(rows < N, h, 0.0)

        @pl.when(i == 0)
        def _():
            s1_out[...] = jnp.zeros_like(s1_out)
            s2_out[...] = jnp.zeros_like(s2_out)
        s1_out[0:1, :] += jnp.sum(h, axis=0, keepdims=True)
        s2_out[0:1, :] += jnp.sum(h * h, axis=0, keepdims=True)

    return pl.pallas_call(
        body,
        grid=(NB,),
        in_specs=[
            pl.BlockSpec((BR, C), lambda i: (i, 0)),
            pl.BlockSpec((BR, C), lambda i: (i, 0)),
            pl.BlockSpec((BR, C), lambda i: (i, 0)),
            pl.BlockSpec((BR, 16), lambda i: (i, 0)),
        ],
        out_specs=[
            pl.BlockSpec((8, 3 * C), lambda i: (0, 0)),
            pl.BlockSpec((8, 3 * C), lambda i: (0, 0)),
        ],
        out_shape=[
            jax.ShapeDtypeStruct((8, 3 * C), jnp.float32),
            jax.ShapeDtypeStruct((8, 3 * C), jnp.float32),
        ],
    )(A, SB, MB, deg)


def _bn_lrelu(h, s1, s2, g, be, N):
    mu = s1[0:1, :] / N
    var = s2[0:1, :] / N - mu * mu
    hn = g * (h - mu) / jnp.sqrt(var + 1e-5) + be
    return jnp.where(hn > 0.0, hn, LRELU * hn)


def _tc_apply(N, A, SB, MB, deg, s1, s2, g, be, wt, wb, b):
    NPAD, C = A.shape
    H = 3 * C
    CO = wt.shape[1]
    BR, NB = _grid_rows(NPAD)

    def body(a_ref, sb_ref, mb_ref, dg_ref, s1_ref, s2_ref, g_ref, be_ref,
             wt_ref, wb_ref, b_ref, an_out, bn_out):
        dg = dg_ref[...][:, 0:1]
        h = _combine(a_ref[...], sb_ref[...], mb_ref[...], dg)
        h = _bn_lrelu(h, s1_ref[...], s2_ref[...], g_ref[...], be_ref[...], N)
        an_out[...] = jnp.dot(h, wt_ref[...],
                              preferred_element_type=jnp.float32) + b_ref[...]
        bn_out[...] = jnp.dot(h, wb_ref[...],
                              preferred_element_type=jnp.float32)

    return pl.pallas_call(
        body,
        grid=(NB,),
        in_specs=[
            pl.BlockSpec((BR, C), lambda i: (i, 0)),
            pl.BlockSpec((BR, C), lambda i: (i, 0)),
            pl.BlockSpec((BR, C), lambda i: (i, 0)),
            pl.BlockSpec((BR, 16), lambda i: (i, 0)),
            pl.BlockSpec((8, H), lambda i: (0, 0)),
            pl.BlockSpec((8, H), lambda i: (0, 0)),
            pl.BlockSpec((1, H), lambda i: (0, 0)),
            pl.BlockSpec((1, H), lambda i: (0, 0)),
            pl.BlockSpec((H, CO), lambda i: (0, 0)),
            pl.BlockSpec((H, CO), lambda i: (0, 0)),
            pl.BlockSpec((1, CO), lambda i: (0, 0)),
        ],
        out_specs=[
            pl.BlockSpec((BR, CO), lambda i: (i, 0)),
            pl.BlockSpec((BR, CO), lambda i: (i, 0)),
        ],
        out_shape=[
            jax.ShapeDtypeStruct((NPAD, CO), jnp.float32),
            jax.ShapeDtypeStruct((NPAD, CO), jnp.float32),
        ],
    )(A, SB, MB, deg, s1, s2, g.reshape(1, H), be.reshape(1, H), wt, wb,
      b.reshape(1, CO))


def _tc_head_a(N, A, SB, MB, deg, s1, s2, g, be, wl1, bl1, wl2, bl2):
    NPAD, C = A.shape
    H = 3 * C
    H1 = wl1.shape[1]
    H2 = wl2.shape[1]
    BR, NB = _grid_rows(NPAD)

    def body(a_ref, sb_ref, mb_ref, dg_ref, s1_ref, s2_ref, g_ref, be_ref,
             w1_ref, b1_ref, w2_ref, b2_ref, a_out, s1_out, s2_out):
        i = pl.program_id(0)
        dg = dg_ref[...][:, 0:1]
        h = _combine(a_ref[...], sb_ref[...], mb_ref[...], dg)
        h = _bn_lrelu(h, s1_ref[...], s2_ref[...], g_ref[...], be_ref[...], N)
        t = jnp.dot(h, w1_ref[...],
                    preferred_element_type=jnp.float32) + b1_ref[...]
        t = jnp.where(t > 0.0, t, LRELU * t)
        a2 = jnp.dot(t, w2_ref[...],
                     preferred_element_type=jnp.float32) + b2_ref[...]
        a_out[...] = a2
        rows = lax.broadcasted_iota(jnp.int32, (BR, 1), 0) + i * BR
        am = jnp.where(rows < N, a2, 0.0)

        @pl.when(i == 0)
        def _():
            s1_out[...] = jnp.zeros_like(s1_out)
            s2_out[...] = jnp.zeros_like(s2_out)
        s1_out[0:1, :] += jnp.sum(am, axis=0, keepdims=True)
        s2_out[0:1, :] += jnp.sum(am * am, axis=0, keepdims=True)

    return pl.pallas_call(
        body,
        grid=(NB,),
        in_specs=[
            pl.BlockSpec((BR, C), lambda i: (i, 0)),
            pl.BlockSpec((BR, C), lambda i: (i, 0)),
            pl.BlockSpec((BR, C), lambda i: (i, 0)),
            pl.BlockSpec((BR, 16), lambda i: (i, 0)),
            pl.BlockSpec((8, H), lambda i: (0, 0)),
            pl.BlockSpec((8, H), lambda i: (0, 0)),
            pl.BlockSpec((1, H), lambda i: (0, 0)),
            pl.BlockSpec((1, H), lambda i: (0, 0)),
            pl.BlockSpec((H, H1), lambda i: (0, 0)),
            pl.BlockSpec((1, H1), lambda i: (0, 0)),
            pl.BlockSpec((H1, H2), lambda i: (0, 0)),
            pl.BlockSpec((1, H2), lambda i: (0, 0)),
        ],
        out_specs=[
            pl.BlockSpec((BR, H2), lambda i: (i, 0)),
            pl.BlockSpec((8, H2), lambda i: (0, 0)),
            pl.BlockSpec((8, H2), lambda i: (0, 0)),
        ],
        out_shape=[
            jax.ShapeDtypeStruct((NPAD, H2), jnp.float32),
            jax.ShapeDtypeStruct((8, H2), jnp.float32),
            jax.ShapeDtypeStruct((8, H2), jnp.float32),
        ],
    )(A, SB, MB, deg, s1, s2, g.reshape(1, H), be.reshape(1, H), wl1,
      bl1.reshape(1, H1), wl2, bl2.reshape(1, H2))


def _tc_head_b(N, a, s1, s2, g4, be4, wo, bo):
    NPAD, H2 = a.shape
    OUT = wo.shape[1]
    WPAD = 128
    BR, NB = _grid_rows(NPAD)
    wo_p = jnp.zeros((H2, WPAD), jnp.float32).at[:, :OUT].set(wo)
    bo_p = jnp.zeros((1, WPAD), jnp.float32).at[0, :OUT].set(bo)

    def body(a_ref, s1_ref, s2_ref, g_ref, be_ref, wo_ref, bo_ref, o_out):
        an = _bn_lrelu(a_ref[...], s1_ref[...], s2_ref[...], g_ref[...],
                       be_ref[...], N)
        o = jnp.dot(an, wo_ref[...],
                    preferred_element_type=jnp.float32) + bo_ref[...]
        cm = lax.broadcasted_iota(jnp.int32, (BR, WPAD), 1) < OUT
        mx = jnp.max(jnp.where(cm, o, -3e38), axis=1, keepdims=True)
        e = jnp.where(cm, jnp.exp(o - mx), 0.0)
        o_out[...] = e / jnp.sum(e, axis=1, keepdims=True)

    return pl.pallas_call(
        body,
        grid=(NB,),
        in_specs=[
            pl.BlockSpec((BR, H2), lambda i: (i, 0)),
            pl.BlockSpec((8, H2), lambda i: (0, 0)),
            pl.BlockSpec((8, H2), lambda i: (0, 0)),
            pl.BlockSpec((1, H2), lambda i: (0, 0)),
            pl.BlockSpec((1, H2), lambda i: (0, 0)),
            pl.BlockSpec((H2, WPAD), lambda i: (0, 0)),
            pl.BlockSpec((1, WPAD), lambda i: (0, 0)),
        ],
        out_specs=pl.BlockSpec((BR, WPAD), lambda i: (i, 0)),
        out_shape=jax.ShapeDtypeStruct((NPAD, WPAD), jnp.float32),
    )(a, s1, s2, g4.reshape(1, H2), be4.reshape(1, H2), wo_p, bo_p)


def kernel(x, edge_index, W1, b1, g1, be1, W2, b2, g2, be2, W3, b3, g3, be3,
           Wl1, bl1, Wl2, bl2, g4, be4, Wo, bo):
    N, F = x.shape
    E = edge_index.shape[1]
    R = -(-N // (NW * 16)) * 16
    NPAD = NW * R
    EB = 4000 if E % 4000 == 0 else E
    C = W1.shape[1]

    xp = jnp.pad(x, ((0, NPAD - N), (0, 0)))

    srcl, dstl, cnts, deg = _make_prep(E, R, EB)(
        edge_index[0].reshape(E), edge_index[1].reshape(E))
    scat = _make_scatter(R, C)

    A, B = _tc_pre(xp, W1[:F], W1[F:], b1)
    SB, MB = scat(B, srcl, dstl, cnts)
    s1, s2 = _tc_stats(N, A, SB, MB, deg)
    A, B = _tc_apply(N, A, SB, MB, deg, s1, s2, g1, be1,
                     W2[:3 * C], W2[3 * C:], b2)

    SB, MB = scat(B, srcl, dstl, cnts)
    s1, s2 = _tc_stats(N, A, SB, MB, deg)
    A, B = _tc_apply(N, A, SB, MB, deg, s1, s2, g2, be2,
                     W3[:3 * C], W3[3 * C:], b3)

    SB, MB = scat(B, srcl, dstl, cnts)
    s1, s2 = _tc_stats(N, A, SB, MB, deg)
    a, h1, h2 = _tc_head_a(N, A, SB, MB, deg, s1, s2, g3, be3,
                           Wl1, bl1, Wl2, bl2)
    out = _tc_head_b(N, a, h1, h2, g4, be4, Wo, bo)
    return out[:N, :Wo.shape[1]]


# speedup vs baseline: 7.8687x; 1.6125x over previous
import functools

import jax
import jax.numpy as jnp
from jax import lax
from jax.experimental import pallas as pl
from jax.experimental.pallas import tpu as pltpu
from jax.experimental.pallas import tpu_sc as plsc

NW = 32
NC, NS, L = 2, 16, 16
G = 128
CAPV = 24576
LRELU = 0.01


def _mesh():
    return plsc.VectorSubcoreMesh(
        core_axis_name="c", subcore_axis_name="s", num_cores=NC,
        num_subcores=NS)


def _wid():
    return lax.axis_index("s") * NC + lax.axis_index("c")


def _make_prep(E, R, EB):
    NPAD = NW * R

    @functools.partial(
        pl.kernel,
        out_type=[
            jax.ShapeDtypeStruct((NW, CAPV), jnp.int32),
            jax.ShapeDtypeStruct((NW, CAPV), jnp.int32),
            jax.ShapeDtypeStruct((NW, 16), jnp.int32),
            jax.ShapeDtypeStruct((NPAD, 16), jnp.float32),
        ],
        mesh=_mesh(),
        scratch_types=[
            pltpu.VMEM((EB,), jnp.int32),
            pltpu.VMEM((EB,), jnp.int32),
            pltpu.VMEM((CAPV + 16,), jnp.int32),
            pltpu.VMEM((CAPV + 16,), jnp.int32),
            pltpu.VMEM((R, 16), jnp.float32),
            pltpu.VMEM((16,), jnp.int32),
        ],
        compiler_params=pltpu.CompilerParams(needs_layout_passes=False),
    )
    def prep(src_hbm, dst_hbm, src_out, dstl_out, cnt_out, deg_out,
             src_b, dst_b, cs_v, cd_v, deg_v, cnt_v):
        wid = _wid()
        lo = wid * R
        zero16 = jnp.zeros((16,), jnp.float32)

        def dinit(i, _):
            deg_v[i, pl.ds(0, 16)] = zero16
            return 0
        lax.fori_loop(0, R, dinit, 0)

        def blk(b, cnt):
            pltpu.sync_copy(src_hbm.at[pl.ds(b * EB, EB)], src_b)
            pltpu.sync_copy(dst_hbm.at[pl.ds(b * EB, EB)], dst_b)

            def vec(i, cnt):
                d = dst_b[pl.ds(i * 16, 16)]
                s = src_b[pl.ds(i * 16, 16)]
                m = (d >= lo) & (d < lo + R)
                pos = plsc.cumsum(m.astype(jnp.int32))
                idx = jnp.minimum(cnt + pos - 1, CAPV - G - 1)
                plsc.store_scatter(cs_v, [idx], s, mask=m)
                plsc.store_scatter(cd_v, [idx], d - lo, mask=m)
                return cnt + jnp.sum(m.astype(jnp.int32))
            return lax.fori_loop(0, EB // 16, vec, cnt)

        cnt = lax.fori_loop(0, E // EB, blk, jnp.int32(0))
        cnt = jnp.minimum(cnt, CAPV - G)

        iot = lax.iota(jnp.int32, 16)
        dump = jnp.full((16,), R + 4, jnp.int32)
        for k in range(G // 16):
            idx = cnt + iot + k * 16
            spread = (wid * 331 + idx * 997) % (NPAD - R)
            plsc.store_scatter(cs_v, [idx], spread, mask=None)
            plsc.store_scatter(cd_v, [idx], dump, mask=None)

        onehot = jnp.where(iot == 0, 1.0, 0.0).astype(jnp.float32)

        def dloop(j, _):
            dl = cd_v[pl.ds(j, 16)][0]
            plsc.addupdate(deg_v.at[dl, pl.ds(0, 16)], onehot)
            return 0
        lax.fori_loop(0, cnt, dloop, 0)

        cnt_v[...] = jnp.full((16,), cnt, jnp.int32)
        pltpu.sync_copy(cs_v.at[pl.ds(0, CAPV)], src_out.at[wid])
        pltpu.sync_copy(cd_v.at[pl.ds(0, CAPV)], dstl_out.at[wid])
        pltpu.sync_copy(cnt_v, cnt_out.at[wid])
        pltpu.sync_copy(deg_v, deg_out.at[pl.ds(lo, R)])

    return prep


def _make_scatter(R, C):
    NPAD = NW * R
    ACC_R = R + 8

    @functools.partial(
        pl.kernel,
        out_type=[
            jax.ShapeDtypeStruct((NPAD, C), jnp.float32),
            jax.ShapeDtypeStruct((NPAD, C), jnp.float32),
        ],
        mesh=_mesh(),
        scratch_types=[
            pltpu.VMEM((CAPV,), jnp.int32),
            pltpu.VMEM((CAPV,), jnp.int32),
            pltpu.VMEM((G, C), jnp.float32),
            pltpu.VMEM((G, C), jnp.float32),
            pltpu.VMEM((ACC_R, C), jnp.float32),
            pltpu.VMEM((ACC_R, C), jnp.float32),
            pltpu.VMEM((16,), jnp.int32),
            pltpu.SemaphoreType.DMA,
            pltpu.SemaphoreType.DMA,
        ],
        compiler_params=pltpu.CompilerParams(
            needs_layout_passes=False, use_tc_tiling_on_sc=False),
    )
    def scat(b_hbm, src_hbm, dstl_hbm, cnt_hbm, sb_out, mb_out,
             slist, dlist, buf0, buf1, accs, accm, cntv, sem0, sem1):
        wid = _wid()
        pltpu.sync_copy(cnt_hbm.at[wid], cntv)
        cnt = cntv[...][0]
        nch = (cnt + G - 1) // G
        pltpu.sync_copy(src_hbm.at[wid], slist)
        pltpu.sync_copy(dstl_hbm.at[wid], dlist)

        zero16 = jnp.zeros((16,), jnp.float32)
        neg16 = jnp.full((16,), -1e30, jnp.float32)

        def init(i, _):
            for k in range(C // 16):
                accs[i, pl.ds(k * 16, 16)] = zero16
                accm[i, pl.ds(k * 16, 16)] = neg16
            return 0
        lax.fori_loop(0, ACC_R, init, 0)

        def start(g, buf, sem):
            pltpu.async_copy(b_hbm.at[slist.at[pl.ds(g * G, G)]], buf, sem)

        def wait(buf, sem):
            pltpu.make_async_copy(b_hbm.at[pl.ds(0, G)], buf, sem).wait()

        def process(g, buf):
            def sub(c, _):
                dl16 = dlist[pl.ds(g * G + c * 16, 16)]
                for j in range(16):
                    dl = dl16[j]
                    r = c * 16 + j
                    for k in range(C // 16):
                        v = buf[r, pl.ds(k * 16, 16)]
                        mx = accm[dl, pl.ds(k * 16, 16)]
                        accm[dl, pl.ds(k * 16, 16)] = jnp.maximum(mx, v)
                        plsc.addupdate(accs.at[dl, pl.ds(k * 16, 16)], v)
                return 0
            lax.fori_loop(0, G // 16, sub, 0)

        @pl.when(nch > 0)
        def _():
            start(0, buf0, sem0)

        def pair(gp, _):
            g0 = 2 * gp
            g1 = g0 + 1

            @pl.when(g1 < nch)
            def _():
                start(g1, buf1, sem1)
            wait(buf0, sem0)
            process(g0, buf0)

            @pl.when(g0 + 2 < nch)
            def _():
                start(g0 + 2, buf0, sem0)

            @pl.when(g1 < nch)
            def _():
                wait(buf1, sem1)
                process(g1, buf1)
            return 0
        lax.fori_loop(0, (nch + 1) // 2, pair, 0)

        pltpu.sync_copy(accs.at[pl.ds(0, R)], sb_out.at[pl.ds(wid * R, R)])
        pltpu.sync_copy(accm.at[pl.ds(0, R)], mb_out.at[pl.ds(wid * R, R)])

    return scat


def _grid_rows(NPAD):
    BR = 1024 if NPAD % 1024 == 0 else NPAD
    return BR, NPAD // BR


def _tc_pre(x, wt, wb, b):
    NPAD, F = x.shape
    C = wt.shape[1]
    BR, NB = _grid_rows(NPAD)

    def body(x_ref, wt_ref, wb_ref, b_ref, a_out, b_out):
        xv = x_ref[...]
        a_out[...] = jnp.dot(xv, wt_ref[...],
                             preferred_element_type=jnp.float32) + b_ref[...]
        b_out[...] = jnp.dot(xv, wb_ref[...],
                             preferred_element_type=jnp.float32)

    return pl.pallas_call(
        body,
        grid=(NB,),
        in_specs=[
            pl.BlockSpec((BR, F), lambda i: (i, 0)),
            pl.BlockSpec((F, C), lambda i: (0, 0)),
            pl.BlockSpec((F, C), lambda i: (0, 0)),
            pl.BlockSpec((1, C), lambda i: (0, 0)),
        ],
        out_specs=[
            pl.BlockSpec((BR, C), lambda i: (i, 0)),
            pl.BlockSpec((BR, C), lambda i: (i, 0)),
        ],
        out_shape=[
            jax.ShapeDtypeStruct((NPAD, C), jnp.float32),
            jax.ShapeDtypeStruct((NPAD, C), jnp.float32),
        ],
    )(x, wt, wb, b.reshape(1, C))


def _combine(A, SB, MB, dg):
    s = dg * A + SB
    mean = s / jnp.maximum(dg, 1.0)
    mx = jnp.where(dg > 0.0, A + MB, 0.0)
    return jnp.concatenate([mx, mean, s], axis=1)


def _tc_stats(N, A, SB, MB, deg):
    NPAD, C = A.shape
    BR, NB = _grid_rows(NPAD)

    def body(a_ref, sb_ref, mb_ref, dg_ref, s1_out, s2_out):
        i = pl.program_id(0)
        dg = dg_ref[...][:, 0:1]
        h = _combine(a_ref[...], sb_ref[...], mb_ref[...], dg)
        rows = lax.broadcasted_iota(jnp.int32, (BR, 1), 0) + i * BR
        h = jnp.where(rows < N, h, 0.0)

        @pl.when(i == 0)
        def _():
            s1_out[...] = jnp.zeros_like(s1_out)
            s2_out[...] = jnp.zeros_like(s2_out)
        s1_out[0:1, :] += jnp.sum(h, axis=0, keepdims=True)
        s2_out[0:1, :] += jnp.sum(h * h, axis=0, keepdims=True)

    return pl.pallas_call(
        body,
        grid=(NB,),
        in_specs=[
            pl.BlockSpec((BR, C), lambda i: (i, 0)),
            pl.BlockSpec((BR, C), lambda i: (i, 0)),
            pl.BlockSpec((BR, C), lambda i: (i, 0)),
            pl.BlockSpec((BR, 16), lambda i: (i, 0)),
        ],
        out_specs=[
            pl.BlockSpec((8, 3 * C), lambda i: (0, 0)),
            pl.BlockSpec((8, 3 * C), lambda i: (0, 0)),
        ],
        out_shape=[
            jax.ShapeDtypeStruct((8, 3 * C), jnp.float32),
            jax.ShapeDtypeStruct((8, 3 * C), jnp.float32),
        ],
    )(A, SB, MB, deg)


def _bn_lrelu(h, s1, s2, g, be, N):
    mu = s1[0:1, :] / N
    var = s2[0:1, :] / N - mu * mu
    hn = g * (h - mu) / jnp.sqrt(var + 1e-5) + be
    return jnp.where(hn > 0.0, hn, LRELU * hn)


def _tc_apply(N, A, SB, MB, deg, s1, s2, g, be, wt, wb, b):
    NPAD, C = A.shape
    H = 3 * C
    CO = wt.shape[1]
    BR, NB = _grid_rows(NPAD)

    def body(a_ref, sb_ref, mb_ref, dg_ref, s1_ref, s2_ref, g_ref, be_ref,
             wt_ref, wb_ref, b_ref, an_out, bn_out):
        dg = dg_ref[...][:, 0:1]
        h = _combine(a_ref[...], sb_ref[...], mb_ref[...], dg)
        h = _bn_lrelu(h, s1_ref[...], s2_ref[...], g_ref[...], be_ref[...], N)
        an_out[...] = jnp.dot(h, wt_ref[...],
                              preferred_element_type=jnp.float32) + b_ref[...]
        bn_out[...] = jnp.dot(h, wb_ref[...],
                              preferred_element_type=jnp.float32)

    return pl.pallas_call(
        body,
        grid=(NB,),
        in_specs=[
            pl.BlockSpec((BR, C), lambda i: (i, 0)),
            pl.BlockSpec((BR, C), lambda i: (i, 0)),
            pl.BlockSpec((BR, C), lambda i: (i, 0)),
            pl.BlockSpec((BR, 16), lambda i: (i, 0)),
            pl.BlockSpec((8, H), lambda i: (0, 0)),
            pl.BlockSpec((8, H), lambda i: (0, 0)),
            pl.BlockSpec((1, H), lambda i: (0, 0)),
            pl.BlockSpec((1, H), lambda i: (0, 0)),
            pl.BlockSpec((H, CO), lambda i: (0, 0)),
            pl.BlockSpec((H, CO), lambda i: (0, 0)),
            pl.BlockSpec((1, CO), lambda i: (0, 0)),
        ],
        out_specs=[
            pl.BlockSpec((BR, CO), lambda i: (i, 0)),
            pl.BlockSpec((BR, CO), lambda i: (i, 0)),
        ],
        out_shape=[
            jax.ShapeDtypeStruct((NPAD, CO), jnp.float32),
            jax.ShapeDtypeStruct((NPAD, CO), jnp.float32),
        ],
    )(A, SB, MB, deg, s1, s2, g.reshape(1, H), be.reshape(1, H), wt, wb,
      b.reshape(1, CO))


def _tc_head_a(N, A, SB, MB, deg, s1, s2, g, be, wl1, bl1, wl2, bl2):
    NPAD, C = A.shape
    H = 3 * C
    H1 = wl1.shape[1]
    H2 = wl2.shape[1]
    BR, NB = _grid_rows(NPAD)

    def body(a_ref, sb_ref, mb_ref, dg_ref, s1_ref, s2_ref, g_ref, be_ref,
             w1_ref, b1_ref, w2_ref, b2_ref, a_out, s1_out, s2_out):
        i = pl.program_id(0)
        dg = dg_ref[...][:, 0:1]
        h = _combine(a_ref[...], sb_ref[...], mb_ref[...], dg)
        h = _bn_lrelu(h, s1_ref[...], s2_ref[...], g_ref[...], be_ref[...], N)
        t = jnp.dot(h, w1_ref[...],
                    preferred_element_type=jnp.float32) + b1_ref[...]
        t = jnp.where(t > 0.0, t, LRELU * t)
        a2 = jnp.dot(t, w2_ref[...],
                     preferred_element_type=jnp.float32) + b2_ref[...]
        a_out[...] = a2
        rows = lax.broadcasted_iota(jnp.int32, (BR, 1), 0) + i * BR
        am = jnp.where(rows < N, a2, 0.0)

        @pl.when(i == 0)
        def _():
            s1_out[...] = jnp.zeros_like(s1_out)
            s2_out[...] = jnp.zeros_like(s2_out)
        s1_out[0:1, :] += jnp.sum(am, axis=0, keepdims=True)
        s2_out[0:1, :] += jnp.sum(am * am, axis=0, keepdims=True)

    return pl.pallas_call(
        body,
        grid=(NB,),
        in_specs=[
            pl.BlockSpec((BR, C), lambda i: (i, 0)),
            pl.BlockSpec((BR, C), lambda i: (i, 0)),
            pl.BlockSpec((BR, C), lambda i: (i, 0)),
            pl.BlockSpec((BR, 16), lambda i: (i, 0)),
            pl.BlockSpec((8, H), lambda i: (0, 0)),
            pl.BlockSpec((8, H), lambda i: (0, 0)),
            pl.BlockSpec((1, H), lambda i: (0, 0)),
            pl.BlockSpec((1, H), lambda i: (0, 0)),
            pl.BlockSpec((H, H1), lambda i: (0, 0)),
            pl.BlockSpec((1, H1), lambda i: (0, 0)),
            pl.BlockSpec((H1, H2), lambda i: (0, 0)),
            pl.BlockSpec((1, H2), lambda i: (0, 0)),
        ],
        out_specs=[
            pl.BlockSpec((BR, H2), lambda i: (i, 0)),
            pl.BlockSpec((8, H2), lambda i: (0, 0)),
            pl.BlockSpec((8, H2), lambda i: (0, 0)),
        ],
        out_shape=[
            jax.ShapeDtypeStruct((NPAD, H2), jnp.float32),
            jax.ShapeDtypeStruct((8, H2), jnp.float32),
            jax.ShapeDtypeStruct((8, H2), jnp.float32),
        ],
    )(A, SB, MB, deg, s1, s2, g.reshape(1, H), be.reshape(1, H), wl1,
      bl1.reshape(1, H1), wl2, bl2.reshape(1, H2))


def _tc_head_b(N, a, s1, s2, g4, be4, wo, bo):
    NPAD, H2 = a.shape
    OUT = wo.shape[1]
    WPAD = 128
    BR, NB = _grid_rows(NPAD)
    wo_p = jnp.zeros((H2, WPAD), jnp.float32).at[:, :OUT].set(wo)
    bo_p = jnp.zeros((1, WPAD), jnp.float32).at[0, :OUT].set(bo)

    def body(a_ref, s1_ref, s2_ref, g_ref, be_ref, wo_ref, bo_ref, o_out):
        an = _bn_lrelu(a_ref[...], s1_ref[...], s2_ref[...], g_ref[...],
                       be_ref[...], N)
        o = jnp.dot(an, wo_ref[...],
                    preferred_element_type=jnp.float32) + bo_ref[...]
        cm = lax.broadcasted_iota(jnp.int32, (BR, WPAD), 1) < OUT
        mx = jnp.max(jnp.where(cm, o, -3e38), axis=1, keepdims=True)
        e = jnp.where(cm, jnp.exp(o - mx), 0.0)
        o_out[...] = e / jnp.sum(e, axis=1, keepdims=True)

    return pl.pallas_call(
        body,
        grid=(NB,),
        in_specs=[
            pl.BlockSpec((BR, H2), lambda i: (i, 0)),
            pl.BlockSpec((8, H2), lambda i: (0, 0)),
            pl.BlockSpec((8, H2), lambda i: (0, 0)),
            pl.BlockSpec((1, H2), lambda i: (0, 0)),
            pl.BlockSpec((1, H2), lambda i: (0, 0)),
            pl.BlockSpec((H2, WPAD), lambda i: (0, 0)),
            pl.BlockSpec((1, WPAD), lambda i: (0, 0)),
        ],
        out_specs=pl.BlockSpec((BR, WPAD), lambda i: (i, 0)),
        out_shape=jax.ShapeDtypeStruct((NPAD, WPAD), jnp.float32),
    )(a, s1, s2, g4.reshape(1, H2), be4.reshape(1, H2), wo_p, bo_p)


def kernel(x, edge_index, W1, b1, g1, be1, W2, b2, g2, be2, W3, b3, g3, be3,
           Wl1, bl1, Wl2, bl2, g4, be4, Wo, bo):
    N, F = x.shape
    E = edge_index.shape[1]
    R = -(-N // (NW * 16)) * 16
    NPAD = NW * R
    EB = 4000 if E % 4000 == 0 else E
    C = W1.shape[1]

    xp = jnp.pad(x, ((0, NPAD - N), (0, 0)))

    srcl, dstl, cnts, deg = _make_prep(E, R, EB)(
        edge_index[0].reshape(E), edge_index[1].reshape(E))
    scat = _make_scatter(R, C)

    A, B = _tc_pre(xp, W1[:F], W1[F:], b1)
    SB, MB = scat(B, srcl, dstl, cnts)
    s1, s2 = _tc_stats(N, A, SB, MB, deg)
    A, B = _tc_apply(N, A, SB, MB, deg, s1, s2, g1, be1,
                     W2[:3 * C], W2[3 * C:], b2)

    SB, MB = scat(B, srcl, dstl, cnts)
    s1, s2 = _tc_stats(N, A, SB, MB, deg)
    A, B = _tc_apply(N, A, SB, MB, deg, s1, s2, g2, be2,
                     W3[:3 * C], W3[3 * C:], b3)

    SB, MB = scat(B, srcl, dstl, cnts)
    s1, s2 = _tc_stats(N, A, SB, MB, deg)
    a, h1, h2 = _tc_head_a(N, A, SB, MB, deg, s1, s2, g3, be3,
                           Wl1, bl1, Wl2, bl2)
    out = _tc_head_b(N, a, h1, h2, g4, be4, Wo, bo)
    return out[:N, :Wo.shape[1]]


# speedup vs baseline: 11.1462x; 1.4165x over previous
import functools

import jax
import jax.numpy as jnp
from jax import lax
from jax.experimental import pallas as pl
from jax.experimental.pallas import tpu as pltpu
from jax.experimental.pallas import tpu_sc as plsc

NW = 32
NC, NS, L = 2, 16, 16
G = 128
CAPV = 24576
LRELU = 0.01


def _mesh():
    return plsc.VectorSubcoreMesh(
        core_axis_name="c", subcore_axis_name="s", num_cores=NC,
        num_subcores=NS)


def _wid():
    return lax.axis_index("s") * NC + lax.axis_index("c")


def _make_prep(E, R, EB):
    NPAD = NW * R
    NB = E // EB
    U = 4

    @functools.partial(
        pl.kernel,
        out_type=[
            jax.ShapeDtypeStruct((NW, CAPV), jnp.int32),
            jax.ShapeDtypeStruct((NW, CAPV), jnp.int32),
            jax.ShapeDtypeStruct((NW, 16), jnp.int32),
            jax.ShapeDtypeStruct((NPAD, 16), jnp.float32),
        ],
        mesh=_mesh(),
        scratch_types=[
            pltpu.VMEM((EB,), jnp.int32),
            pltpu.VMEM((EB,), jnp.int32),
            pltpu.VMEM((EB,), jnp.int32),
            pltpu.VMEM((EB,), jnp.int32),
            pltpu.VMEM((CAPV,), jnp.int32),
            pltpu.VMEM((CAPV,), jnp.int32),
            pltpu.VMEM((R + 8, 16), jnp.float32),
            pltpu.VMEM((16,), jnp.int32),
            pltpu.SemaphoreType.DMA,
            pltpu.SemaphoreType.DMA,
        ],
        compiler_params=pltpu.CompilerParams(needs_layout_passes=False),
    )
    def prep(src_hbm, dst_hbm, src_out, dstl_out, cnt_out, deg_out,
             sb0, db0, sb1, db1, cs_v, cd_v, deg_v, cnt_v, sem0, sem1):
        wid = _wid()
        lo = wid * R
        zero16 = jnp.zeros((16,), jnp.float32)
        one16i = jnp.ones((16,), jnp.int32)
        zero16i = jnp.zeros((16,), jnp.int32)

        def dinit(i, _):
            deg_v[i, pl.ds(0, 16)] = zero16
            return 0
        lax.fori_loop(0, R + 8, dinit, 0)

        def start(b, sb, db, sem):
            pltpu.async_copy(src_hbm.at[pl.ds(b * EB, EB)], sb, sem)
            pltpu.async_copy(dst_hbm.at[pl.ds(b * EB, EB)], db, sem)

        def wait(sb, db, sem):
            pltpu.make_async_copy(src_hbm.at[pl.ds(0, EB)], sb, sem).wait()
            pltpu.make_async_copy(src_hbm.at[pl.ds(0, EB)], db, sem).wait()

        def filt(sb, db, cnt):
            def vec(i, cnt):
                ds_, ss_, ms_, ps_ = [], [], [], []
                for u in range(U):
                    off = (i * U + u) * 16
                    d = db[pl.ds(off, 16)]
                    s = sb[pl.ds(off, 16)]
                    m = (d >= lo) & (d < lo + R)
                    pos = plsc.cumsum(jnp.where(m, one16i, zero16i))
                    ds_.append(d)
                    ss_.append(s)
                    ms_.append(m)
                    ps_.append(pos)
                for u in range(U):
                    idx = jnp.minimum(cnt + ps_[u] - 1, CAPV - G - 1)
                    plsc.store_scatter(cs_v, [idx], ss_[u], mask=ms_[u])
                    plsc.store_scatter(cd_v, [idx], ds_[u] - lo, mask=ms_[u])
                    cnt = cnt + ps_[u][15]
                return cnt
            return lax.fori_loop(0, EB // (16 * U), vec, cnt)

        start(0, sb0, db0, sem0)

        def pair(bp, cnt):
            b0 = 2 * bp
            start(b0 + 1, sb1, db1, sem1)
            wait(sb0, db0, sem0)
            cnt = filt(sb0, db0, cnt)

            @pl.when(b0 + 2 < NB)
            def _():
                start(b0 + 2, sb0, db0, sem0)
            wait(sb1, db1, sem1)
            return filt(sb1, db1, cnt)
        cnt = lax.fori_loop(0, NB // 2, pair, jnp.int32(0))
        cnt = jnp.minimum(cnt, CAPV - G)

        iot = lax.iota(jnp.int32, 16)
        dump = jnp.full((16,), R + 4, jnp.int32)
        for k in range(G // 16):
            idx = cnt + iot + k * 16
            spread = (wid * 331 + idx * 997) % (NPAD - R)
            plsc.store_scatter(cs_v, [idx], spread, mask=None)
            plsc.store_scatter(cd_v, [idx], dump, mask=None)

        onehot = jnp.where(iot == 0, 1.0, 0.0)

        def dgrp(c, _):
            dl16 = cd_v[pl.ds(c * 16, 16)]
            for j in range(16):
                plsc.addupdate(deg_v.at[dl16[j], pl.ds(0, 16)], onehot)
            return 0
        lax.fori_loop(0, (cnt + 15) // 16, dgrp, 0)

        cnt_v[...] = jnp.full((16,), cnt, jnp.int32)
        pltpu.sync_copy(cs_v, src_out.at[wid])
        pltpu.sync_copy(cd_v, dstl_out.at[wid])
        pltpu.sync_copy(cnt_v, cnt_out.at[wid])
        pltpu.sync_copy(deg_v.at[pl.ds(0, R)], deg_out.at[pl.ds(lo, R)])

    return prep


def _make_scatter(R, C):
    NPAD = NW * R
    ACC_R = R + 8

    @functools.partial(
        pl.kernel,
        out_type=[
            jax.ShapeDtypeStruct((NPAD, C), jnp.float32),
            jax.ShapeDtypeStruct((NPAD, C), jnp.float32),
        ],
        mesh=_mesh(),
        scratch_types=[
            pltpu.VMEM((CAPV,), jnp.int32),
            pltpu.VMEM((CAPV,), jnp.int32),
            pltpu.VMEM((G, C), jnp.float32),
            pltpu.VMEM((G, C), jnp.float32),
            pltpu.VMEM((ACC_R, C), jnp.float32),
            pltpu.VMEM((ACC_R, C), jnp.float32),
            pltpu.VMEM((16,), jnp.int32),
            pltpu.SemaphoreType.DMA,
            pltpu.SemaphoreType.DMA,
        ],
        compiler_params=pltpu.CompilerParams(
            needs_layout_passes=False, use_tc_tiling_on_sc=False),
    )
    def scat(b_hbm, src_hbm, dstl_hbm, cnt_hbm, sb_out, mb_out,
             slist, dlist, buf0, buf1, accs, accm, cntv, sem0, sem1):
        wid = _wid()
        pltpu.sync_copy(cnt_hbm.at[wid], cntv)
        cnt = cntv[...][0]
        nch = (cnt + G - 1) // G
        pltpu.sync_copy(src_hbm.at[wid], slist)
        pltpu.sync_copy(dstl_hbm.at[wid], dlist)

        zero16 = jnp.zeros((16,), jnp.float32)
        neg16 = jnp.full((16,), -1e30, jnp.float32)

        def init(i, _):
            for k in range(C // 16):
                accs[i, pl.ds(k * 16, 16)] = zero16
                accm[i, pl.ds(k * 16, 16)] = neg16
            return 0
        lax.fori_loop(0, ACC_R, init, 0)

        def start(g, buf, sem):
            pltpu.async_copy(b_hbm.at[slist.at[pl.ds(g * G, G)]], buf, sem)

        def wait(buf, sem):
            pltpu.make_async_copy(b_hbm.at[pl.ds(0, G)], buf, sem).wait()

        def process(g, buf):
            def sub(c, _):
                dl16 = dlist[pl.ds(g * G + c * 16, 16)]
                for j in range(16):
                    dl = dl16[j]
                    r = c * 16 + j
                    for k in range(C // 16):
                        v = buf[r, pl.ds(k * 16, 16)]
                        mx = accm[dl, pl.ds(k * 16, 16)]
                        accm[dl, pl.ds(k * 16, 16)] = jnp.maximum(mx, v)
                        plsc.addupdate(accs.at[dl, pl.ds(k * 16, 16)], v)
                return 0
            lax.fori_loop(0, G // 16, sub, 0)

        @pl.when(nch > 0)
        def _():
            start(0, buf0, sem0)

        def pair(gp, _):
            g0 = 2 * gp
            g1 = g0 + 1

            @pl.when(g1 < nch)
            def _():
                start(g1, buf1, sem1)
            wait(buf0, sem0)
            process(g0, buf0)

            @pl.when(g0 + 2 < nch)
            def _():
                start(g0 + 2, buf0, sem0)

            @pl.when(g1 < nch)
            def _():
                wait(buf1, sem1)
                process(g1, buf1)
            return 0
        lax.fori_loop(0, (nch + 1) // 2, pair, 0)

        pltpu.sync_copy(accs.at[pl.ds(0, R)], sb_out.at[pl.ds(wid * R, R)])
        pltpu.sync_copy(accm.at[pl.ds(0, R)], mb_out.at[pl.ds(wid * R, R)])

    return scat


def _grid_rows(NPAD):
    BR = 1024 if NPAD % 1024 == 0 else NPAD
    return BR, NPAD // BR


def _tc_pre(x, wt, wb, b):
    NPAD, F = x.shape
    C = wt.shape[1]
    BR, NB = _grid_rows(NPAD)

    def body(x_ref, wt_ref, wb_ref, b_ref, a_out, b_out):
        xv = x_ref[...]
        a_out[...] = jnp.dot(xv, wt_ref[...],
                             preferred_element_type=jnp.float32) + b_ref[...]
        b_out[...] = jnp.dot(xv, wb_ref[...],
                             preferred_element_type=jnp.float32)

    return pl.pallas_call(
        body,
        grid=(NB,),
        in_specs=[
            pl.BlockSpec((BR, F), lambda i: (i, 0)),
            pl.BlockSpec((F, C), lambda i: (0, 0)),
            pl.BlockSpec((F, C), lambda i: (0, 0)),
            pl.BlockSpec((1, C), lambda i: (0, 0)),
        ],
        out_specs=[
            pl.BlockSpec((BR, C), lambda i: (i, 0)),
            pl.BlockSpec((BR, C), lambda i: (i, 0)),
        ],
        out_shape=[
            jax.ShapeDtypeStruct((NPAD, C), jnp.float32),
            jax.ShapeDtypeStruct((NPAD, C), jnp.float32),
        ],
    )(x, wt, wb, b.reshape(1, C))


def _combine(A, SB, MB, dg):
    s = dg * A + SB
    mean = s / jnp.maximum(dg, 1.0)
    mx = jnp.where(dg > 0.0, A + MB, 0.0)
    return jnp.concatenate([mx, mean, s], axis=1)


def _tc_stats(N, A, SB, MB, deg):
    NPAD, C = A.shape
    BR, NB = _grid_rows(NPAD)

    def body(a_ref, sb_ref, mb_ref, dg_ref, s1_out, s2_out):
        i = pl.program_id(0)
        dg = dg_ref[...][:, 0:1]
        h = _combine(a_ref[...], sb_ref[...], mb_ref[...], dg)
        rows = lax.broadcasted_iota(jnp.int32, (BR, 1), 0) + i * BR
        h = jnp.where(rows < N, h, 0.0)

        @pl.when(i == 0)
        def _():
            s1_out[...] = jnp.zeros_like(s1_out)
            s2_out[...] = jnp.zeros_like(s2_out)
        s1_out[0:1, :] += jnp.sum(h, axis=0, keepdims=True)
        s2_out[0:1, :] += jnp.sum(h * h, axis=0, keepdims=True)

    return pl.pallas_call(
        body,
        grid=(NB,),
        in_specs=[
            pl.BlockSpec((BR, C), lambda i: (i, 0)),
            pl.BlockSpec((BR, C), lambda i: (i, 0)),
            pl.BlockSpec((BR, C), lambda i: (i, 0)),
            pl.BlockSpec((BR, 16), lambda i: (i, 0)),
        ],
        out_specs=[
            pl.BlockSpec((8, 3 * C), lambda i: (0, 0)),
            pl.BlockSpec((8, 3 * C), lambda i: (0, 0)),
        ],
        out_shape=[
            jax.ShapeDtypeStruct((8, 3 * C), jnp.float32),
            jax.ShapeDtypeStruct((8, 3 * C), jnp.float32),
        ],
    )(A, SB, MB, deg)


def _bn_lrelu(h, s1, s2, g, be, N):
    mu = s1[0:1, :] / N
    var = s2[0:1, :] / N - mu * mu
    hn = g * (h - mu) / jnp.sqrt(var + 1e-5) + be
    return jnp.where(hn > 0.0, hn, LRELU * hn)


def _tc_apply(N, A, SB, MB, deg, s1, s2, g, be, wt, wb, b):
    NPAD, C = A.shape
    H = 3 * C
    CO = wt.shape[1]
    BR, NB = _grid_rows(NPAD)

    def body(a_ref, sb_ref, mb_ref, dg_ref, s1_ref, s2_ref, g_ref, be_ref,
             wt_ref, wb_ref, b_ref, an_out, bn_out):
        dg = dg_ref[...][:, 0:1]
        h = _combine(a_ref[...], sb_ref[...], mb_ref[...], dg)
        h = _bn_lrelu(h, s1_ref[...], s2_ref[...], g_ref[...], be_ref[...], N)
        an_out[...] = jnp.dot(h, wt_ref[...],
                              preferred_element_type=jnp.float32) + b_ref[...]
        bn_out[...] = jnp.dot(h, wb_ref[...],
                              preferred_element_type=jnp.float32)

    return pl.pallas_call(
        body,
        grid=(NB,),
        in_specs=[
            pl.BlockSpec((BR, C), lambda i: (i, 0)),
            pl.BlockSpec((BR, C), lambda i: (i, 0)),
            pl.BlockSpec((BR, C), lambda i: (i, 0)),
            pl.BlockSpec((BR, 16), lambda i: (i, 0)),
            pl.BlockSpec((8, H), lambda i: (0, 0)),
            pl.BlockSpec((8, H), lambda i: (0, 0)),
            pl.BlockSpec((1, H), lambda i: (0, 0)),
            pl.BlockSpec((1, H), lambda i: (0, 0)),
            pl.BlockSpec((H, CO), lambda i: (0, 0)),
            pl.BlockSpec((H, CO), lambda i: (0, 0)),
            pl.BlockSpec((1, CO), lambda i: (0, 0)),
        ],
        out_specs=[
            pl.BlockSpec((BR, CO), lambda i: (i, 0)),
            pl.BlockSpec((BR, CO), lambda i: (i, 0)),
        ],
        out_shape=[
            jax.ShapeDtypeStruct((NPAD, CO), jnp.float32),
            jax.ShapeDtypeStruct((NPAD, CO), jnp.float32),
        ],
    )(A, SB, MB, deg, s1, s2, g.reshape(1, H), be.reshape(1, H), wt, wb,
      b.reshape(1, CO))


def _tc_head_a(N, A, SB, MB, deg, s1, s2, g, be, wl1, bl1, wl2, bl2):
    NPAD, C = A.shape
    H = 3 * C
    H1 = wl1.shape[1]
    H2 = wl2.shape[1]
    BR, NB = _grid_rows(NPAD)

    def body(a_ref, sb_ref, mb_ref, dg_ref, s1_ref, s2_ref, g_ref, be_ref,
             w1_ref, b1_ref, w2_ref, b2_ref, a_out, s1_out, s2_out):
        i = pl.program_id(0)
        dg = dg_ref[...][:, 0:1]
        h = _combine(a_ref[...], sb_ref[...], mb_ref[...], dg)
        h = _bn_lrelu(h, s1_ref[...], s2_ref[...], g_ref[...], be_ref[...], N)
        t = jnp.dot(h, w1_ref[...],
                    preferred_element_type=jnp.float32) + b1_ref[...]
        t = jnp.where(t > 0.0, t, LRELU * t)
        a2 = jnp.dot(t, w2_ref[...],
                     preferred_element_type=jnp.float32) + b2_ref[...]
        a_out[...] = a2
        rows = lax.broadcasted_iota(jnp.int32, (BR, 1), 0) + i * BR
        am = jnp.where(rows < N, a2, 0.0)

        @pl.when(i == 0)
        def _():
            s1_out[...] = jnp.zeros_like(s1_out)
            s2_out[...] = jnp.zeros_like(s2_out)
        s1_out[0:1, :] += jnp.sum(am, axis=0, keepdims=True)
        s2_out[0:1, :] += jnp.sum(am * am, axis=0, keepdims=True)

    return pl.pallas_call(
        body,
        grid=(NB,),
        in_specs=[
            pl.BlockSpec((BR, C), lambda i: (i, 0)),
            pl.BlockSpec((BR, C), lambda i: (i, 0)),
            pl.BlockSpec((BR, C), lambda i: (i, 0)),
            pl.BlockSpec((BR, 16), lambda i: (i, 0)),
            pl.BlockSpec((8, H), lambda i: (0, 0)),
            pl.BlockSpec((8, H), lambda i: (0, 0)),
            pl.BlockSpec((1, H), lambda i: (0, 0)),
            pl.BlockSpec((1, H), lambda i: (0, 0)),
            pl.BlockSpec((H, H1), lambda i: (0, 0)),
            pl.BlockSpec((1, H1), lambda i: (0, 0)),
            pl.BlockSpec((H1, H2), lambda i: (0, 0)),
            pl.BlockSpec((1, H2), lambda i: (0, 0)),
        ],
        out_specs=[
            pl.BlockSpec((BR, H2), lambda i: (i, 0)),
            pl.BlockSpec((8, H2), lambda i: (0, 0)),
            pl.BlockSpec((8, H2), lambda i: (0, 0)),
        ],
        out_shape=[
            jax.ShapeDtypeStruct((NPAD, H2), jnp.float32),
            jax.ShapeDtypeStruct((8, H2), jnp.float32),
            jax.ShapeDtypeStruct((8, H2), jnp.float32),
        ],
    )(A, SB, MB, deg, s1, s2, g.reshape(1, H), be.reshape(1, H), wl1,
      bl1.reshape(1, H1), wl2, bl2.reshape(1, H2))


def _tc_head_b(N, a, s1, s2, g4, be4, wo, bo):
    NPAD, H2 = a.shape
    OUT = wo.shape[1]
    WPAD = 128
    BR, NB = _grid_rows(NPAD)
    wo_p = jnp.zeros((H2, WPAD), jnp.float32).at[:, :OUT].set(wo)
    bo_p = jnp.zeros((1, WPAD), jnp.float32).at[0, :OUT].set(bo)

    def body(a_ref, s1_ref, s2_ref, g_ref, be_ref, wo_ref, bo_ref, o_out):
        an = _bn_lrelu(a_ref[...], s1_ref[...], s2_ref[...], g_ref[...],
                       be_ref[...], N)
        o = jnp.dot(an, wo_ref[...],
                    preferred_element_type=jnp.float32) + bo_ref[...]
        cm = lax.broadcasted_iota(jnp.int32, (BR, WPAD), 1) < OUT
        mx = jnp.max(jnp.where(cm, o, -3e38), axis=1, keepdims=True)
        e = jnp.where(cm, jnp.exp(o - mx), 0.0)
        o_out[...] = e / jnp.sum(e, axis=1, keepdims=True)

    return pl.pallas_call(
        body,
        grid=(NB,),
        in_specs=[
            pl.BlockSpec((BR, H2), lambda i: (i, 0)),
            pl.BlockSpec((8, H2), lambda i: (0, 0)),
            pl.BlockSpec((8, H2), lambda i: (0, 0)),
            pl.BlockSpec((1, H2), lambda i: (0, 0)),
            pl.BlockSpec((1, H2), lambda i: (0, 0)),
            pl.BlockSpec((H2, WPAD), lambda i: (0, 0)),
            pl.BlockSpec((1, WPAD), lambda i: (0, 0)),
        ],
        out_specs=pl.BlockSpec((BR, WPAD), lambda i: (i, 0)),
        out_shape=jax.ShapeDtypeStruct((NPAD, WPAD), jnp.float32),
    )(a, s1, s2, g4.reshape(1, H2), be4.reshape(1, H2), wo_p, bo_p)


def kernel(x, edge_index, W1, b1, g1, be1, W2, b2, g2, be2, W3, b3, g3, be3,
           Wl1, bl1, Wl2, bl2, g4, be4, Wo, bo):
    N, F = x.shape
    E = edge_index.shape[1]
    R = -(-N // (NW * 16)) * 16
    NPAD = NW * R
    EB = 6400 if E % 6400 == 0 else E
    C = W1.shape[1]

    xp = jnp.pad(x, ((0, NPAD - N), (0, 0)))

    srcl, dstl, cnts, deg = _make_prep(E, R, EB)(
        edge_index[0].reshape(E), edge_index[1].reshape(E))
    scat = _make_scatter(R, C)

    A, B = _tc_pre(xp, W1[:F], W1[F:], b1)
    SB, MB = scat(B, srcl, dstl, cnts)
    s1, s2 = _tc_stats(N, A, SB, MB, deg)
    A, B = _tc_apply(N, A, SB, MB, deg, s1, s2, g1, be1,
                     W2[:3 * C], W2[3 * C:], b2)

    SB, MB = scat(B, srcl, dstl, cnts)
    s1, s2 = _tc_stats(N, A, SB, MB, deg)
    A, B = _tc_apply(N, A, SB, MB, deg, s1, s2, g2, be2,
                     W3[:3 * C], W3[3 * C:], b3)

    SB, MB = scat(B, srcl, dstl, cnts)
    s1, s2 = _tc_stats(N, A, SB, MB, deg)
    a, h1, h2 = _tc_head_a(N, A, SB, MB, deg, s1, s2, g3, be3,
                           Wl1, bl1, Wl2, bl2)
    out = _tc_head_b(N, a, h1, h2, g4, be4, Wo, bo)
    return out[:N, :Wo.shape[1]]


# speedup vs baseline: 14.9175x; 1.3383x over previous
import functools

import jax
import jax.numpy as jnp
from jax import lax
from jax.experimental import pallas as pl
from jax.experimental.pallas import tpu as pltpu
from jax.experimental.pallas import tpu_sc as plsc

NW = 32
NC, NS, L = 2, 16, 16
G = 128
CAPV = 24576
LRELU = 0.01


def _mesh():
    return plsc.VectorSubcoreMesh(
        core_axis_name="c", subcore_axis_name="s", num_cores=NC,
        num_subcores=NS)


def _wid():
    return lax.axis_index("s") * NC + lax.axis_index("c")


def _make_prep(E, R, EB):
    NPAD = NW * R
    NB = E // EB
    U = 4

    @functools.partial(
        pl.kernel,
        out_type=[
            jax.ShapeDtypeStruct((NW, CAPV), jnp.int32),
            jax.ShapeDtypeStruct((NW, CAPV), jnp.int32),
            jax.ShapeDtypeStruct((NW, 16), jnp.int32),
            jax.ShapeDtypeStruct((NPAD, 16), jnp.float32),
        ],
        mesh=_mesh(),
        scratch_types=[
            pltpu.VMEM((EB,), jnp.int32),
            pltpu.VMEM((EB,), jnp.int32),
            pltpu.VMEM((EB,), jnp.int32),
            pltpu.VMEM((EB,), jnp.int32),
            pltpu.VMEM((CAPV,), jnp.int32),
            pltpu.VMEM((CAPV,), jnp.int32),
            pltpu.VMEM((R + 8, 16), jnp.float32),
            pltpu.VMEM((16,), jnp.int32),
            pltpu.SemaphoreType.DMA,
            pltpu.SemaphoreType.DMA,
        ],
        compiler_params=pltpu.CompilerParams(needs_layout_passes=False),
    )
    def prep(src_hbm, dst_hbm, src_out, dstl_out, cnt_out, deg_out,
             sb0, db0, sb1, db1, cs_v, cd_v, deg_v, cnt_v, sem0, sem1):
        wid = _wid()
        lo = wid * R
        zero16 = jnp.zeros((16,), jnp.float32)
        one16i = jnp.ones((16,), jnp.int32)
        zero16i = jnp.zeros((16,), jnp.int32)

        def dinit(i, _):
            deg_v[i, pl.ds(0, 16)] = zero16
            return 0
        lax.fori_loop(0, R + 8, dinit, 0)

        def start(b, sb, db, sem):
            pltpu.async_copy(src_hbm.at[pl.ds(b * EB, EB)], sb, sem)
            pltpu.async_copy(dst_hbm.at[pl.ds(b * EB, EB)], db, sem)

        def wait(sb, db, sem):
            pltpu.make_async_copy(src_hbm.at[pl.ds(0, EB)], sb, sem).wait()
            pltpu.make_async_copy(src_hbm.at[pl.ds(0, EB)], db, sem).wait()

        def filt(sb, db, cnt):
            def vec(i, cnt):
                ds_, ss_, ms_, ps_ = [], [], [], []
                for u in range(U):
                    off = (i * U + u) * 16
                    d = db[pl.ds(off, 16)]
                    s = sb[pl.ds(off, 16)]
                    m = (d >= lo) & (d < lo + R)
                    pos = plsc.cumsum(jnp.where(m, one16i, zero16i))
                    ds_.append(d)
                    ss_.append(s)
                    ms_.append(m)
                    ps_.append(pos)
                for u in range(U):
                    idx = jnp.minimum(cnt + ps_[u] - 1, CAPV - G - 1)
                    plsc.store_scatter(cs_v, [idx], ss_[u], mask=ms_[u])
                    plsc.store_scatter(cd_v, [idx], ds_[u] - lo, mask=ms_[u])
                    cnt = cnt + ps_[u][15]
                return cnt
            return lax.fori_loop(0, EB // (16 * U), vec, cnt)

        start(0, sb0, db0, sem0)

        def pair(bp, cnt):
            b0 = 2 * bp
            start(b0 + 1, sb1, db1, sem1)
            wait(sb0, db0, sem0)
            cnt = filt(sb0, db0, cnt)

            @pl.when(b0 + 2 < NB)
            def _():
                start(b0 + 2, sb0, db0, sem0)
            wait(sb1, db1, sem1)
            return filt(sb1, db1, cnt)
        cnt = lax.fori_loop(0, NB // 2, pair, jnp.int32(0))
        cnt = jnp.minimum(cnt, CAPV - G)

        iot = lax.iota(jnp.int32, 16)
        dump = jnp.full((16,), R + 4, jnp.int32)
        for k in range(G // 16):
            idx = cnt + iot + k * 16
            spread = (wid * 331 + idx * 997) % (NPAD - R)
            plsc.store_scatter(cs_v, [idx], spread, mask=None)
            plsc.store_scatter(cd_v, [idx], dump, mask=None)

        onehot = jnp.where(iot == 0, 1.0, 0.0)

        def dgrp(c, _):
            dl16 = cd_v[pl.ds(c * 16, 16)]
            for j in range(16):
                plsc.addupdate(deg_v.at[dl16[j], pl.ds(0, 16)], onehot)
            return 0
        lax.fori_loop(0, (cnt + 15) // 16, dgrp, 0)

        cnt_v[...] = jnp.full((16,), cnt, jnp.int32)
        pltpu.sync_copy(cs_v, src_out.at[wid])
        pltpu.sync_copy(cd_v, dstl_out.at[wid])
        pltpu.sync_copy(cnt_v, cnt_out.at[wid])
        pltpu.sync_copy(deg_v.at[pl.ds(0, R)], deg_out.at[pl.ds(lo, R)])

    return prep


def _make_scatter(R, C):
    NPAD = NW * R
    ACC_R = R + 8

    @functools.partial(
        pl.kernel,
        out_type=[
            jax.ShapeDtypeStruct((NPAD, C), jnp.float32),
            jax.ShapeDtypeStruct((NPAD, C), jnp.float32),
        ],
        mesh=_mesh(),
        scratch_types=[
            pltpu.VMEM((CAPV,), jnp.int32),
            pltpu.VMEM((CAPV,), jnp.int32),
            pltpu.VMEM((G, C), jnp.float32),
            pltpu.VMEM((G, C), jnp.float32),
            pltpu.VMEM((ACC_R, C), jnp.float32),
            pltpu.VMEM((ACC_R, C), jnp.float32),
            pltpu.VMEM((16,), jnp.int32),
            pltpu.SemaphoreType.DMA,
            pltpu.SemaphoreType.DMA,
        ],
        compiler_params=pltpu.CompilerParams(
            needs_layout_passes=False, use_tc_tiling_on_sc=False),
    )
    def scat(b_hbm, src_hbm, dstl_hbm, cnt_hbm, sb_out, mb_out,
             slist, dlist, buf0, buf1, accs, accm, cntv, sem0, sem1):
        wid = _wid()
        pltpu.sync_copy(cnt_hbm.at[wid], cntv)
        cnt = cntv[...][0]
        nch = (cnt + G - 1) // G
        pltpu.sync_copy(src_hbm.at[wid], slist)
        pltpu.sync_copy(dstl_hbm.at[wid], dlist)

        zero16 = jnp.zeros((16,), jnp.float32)
        neg16 = jnp.full((16,), -1e30, jnp.float32)

        def init(i, _):
            for k in range(C // 16):
                accs[i, pl.ds(k * 16, 16)] = zero16
                accm[i, pl.ds(k * 16, 16)] = neg16
            return 0
        lax.fori_loop(0, ACC_R, init, 0)

        def start(g, buf, sem):
            pltpu.async_copy(b_hbm.at[slist.at[pl.ds(g * G, G)]], buf, sem)

        def wait(buf, sem):
            pltpu.make_async_copy(b_hbm.at[pl.ds(0, G)], buf, sem).wait()

        def process(g, buf):
            def sub(c, _):
                dl16 = dlist[pl.ds(g * G + c * 16, 16)]
                for j in range(16):
                    dl = dl16[j]
                    r = c * 16 + j
                    vs = [buf[r, pl.ds(k * 16, 16)] for k in range(C // 16)]
                    ms = [accm[dl, pl.ds(k * 16, 16)]
                          for k in range(C // 16)]
                    for k in range(C // 16):
                        accm[dl, pl.ds(k * 16, 16)] = jnp.maximum(ms[k],
                                                                  vs[k])
                        plsc.addupdate(accs.at[dl, pl.ds(k * 16, 16)], vs[k])
                return 0
            lax.fori_loop(0, G // 16, sub, 0)

        @pl.when(nch > 0)
        def _():
            start(0, buf0, sem0)

        def pair(gp, _):
            g0 = 2 * gp
            g1 = g0 + 1

            @pl.when(g1 < nch)
            def _():
                start(g1, buf1, sem1)
            wait(buf0, sem0)
            process(g0, buf0)

            @pl.when(g0 + 2 < nch)
            def _():
                start(g0 + 2, buf0, sem0)

            @pl.when(g1 < nch)
            def _():
                wait(buf1, sem1)
                process(g1, buf1)
            return 0
        lax.fori_loop(0, (nch + 1) // 2, pair, 0)

        pltpu.sync_copy(accs.at[pl.ds(0, R)], sb_out.at[pl.ds(wid * R, R)])
        pltpu.sync_copy(accm.at[pl.ds(0, R)], mb_out.at[pl.ds(wid * R, R)])

    return scat


def _grid_rows(NPAD):
    BR = 1024 if NPAD % 1024 == 0 else NPAD
    return BR, NPAD // BR


def _tc_pre(x, wt, wb, b):
    NPAD, F = x.shape
    C = wt.shape[1]
    BR, NB = _grid_rows(NPAD)

    def body(x_ref, wt_ref, wb_ref, b_ref, a_out, b_out):
        xv = x_ref[...]
        a_out[...] = jnp.dot(xv, wt_ref[...],
                             preferred_element_type=jnp.float32) + b_ref[...]
        b_out[...] = jnp.dot(xv, wb_ref[...],
                             preferred_element_type=jnp.float32)

    return pl.pallas_call(
        body,
        grid=(NB,),
        in_specs=[
            pl.BlockSpec((BR, F), lambda i: (i, 0)),
            pl.BlockSpec((F, C), lambda i: (0, 0)),
            pl.BlockSpec((F, C), lambda i: (0, 0)),
            pl.BlockSpec((1, C), lambda i: (0, 0)),
        ],
        out_specs=[
            pl.BlockSpec((BR, C), lambda i: (i, 0)),
            pl.BlockSpec((BR, C), lambda i: (i, 0)),
        ],
        out_shape=[
            jax.ShapeDtypeStruct((NPAD, C), jnp.float32),
            jax.ShapeDtypeStruct((NPAD, C), jnp.float32),
        ],
    )(x, wt, wb, b.reshape(1, C))


def _combine(A, SB, MB, dg):
    s = dg * A + SB
    mean = s / jnp.maximum(dg, 1.0)
    mx = jnp.where(dg > 0.0, A + MB, 0.0)
    return jnp.concatenate([mx, mean, s], axis=1)


def _tc_stats(N, A, SB, MB, deg):
    NPAD, C = A.shape
    BR, NB = _grid_rows(NPAD)

    def body(a_ref, sb_ref, mb_ref, dg_ref, s1_out, s2_out):
        i = pl.program_id(0)
        dg = dg_ref[...][:, 0:1]
        h = _combine(a_ref[...], sb_ref[...], mb_ref[...], dg)
        rows = lax.broadcasted_iota(jnp.int32, (BR, 1), 0) + i * BR
        h = jnp.where(rows < N, h, 0.0)

        @pl.when(i == 0)
        def _():
            s1_out[...] = jnp.zeros_like(s1_out)
            s2_out[...] = jnp.zeros_like(s2_out)
        s1_out[0:1, :] += jnp.sum(h, axis=0, keepdims=True)
        s2_out[0:1, :] += jnp.sum(h * h, axis=0, keepdims=True)

    return pl.pallas_call(
        body,
        grid=(NB,),
        in_specs=[
            pl.BlockSpec((BR, C), lambda i: (i, 0)),
            pl.BlockSpec((BR, C), lambda i: (i, 0)),
            pl.BlockSpec((BR, C), lambda i: (i, 0)),
            pl.BlockSpec((BR, 16), lambda i: (i, 0)),
        ],
        out_specs=[
            pl.BlockSpec((8, 3 * C), lambda i: (0, 0)),
            pl.BlockSpec((8, 3 * C), lambda i: (0, 0)),
        ],
        out_shape=[
            jax.ShapeDtypeStruct((8, 3 * C), jnp.float32),
            jax.ShapeDtypeStruct((8, 3 * C), jnp.float32),
        ],
    )(A, SB, MB, deg)


def _bn_lrelu(h, s1, s2, g, be, N):
    mu = s1[0:1, :] / N
    var = s2[0:1, :] / N - mu * mu
    hn = g * (h - mu) / jnp.sqrt(var + 1e-5) + be
    return jnp.where(hn > 0.0, hn, LRELU * hn)


def _tc_apply(N, A, SB, MB, deg, s1, s2, g, be, wt, wb, b):
    NPAD, C = A.shape
    H = 3 * C
    CO = wt.shape[1]
    BR, NB = _grid_rows(NPAD)

    def body(a_ref, sb_ref, mb_ref, dg_ref, s1_ref, s2_ref, g_ref, be_ref,
             wt_ref, wb_ref, b_ref, an_out, bn_out):
        dg = dg_ref[...][:, 0:1]
        h = _combine(a_ref[...], sb_ref[...], mb_ref[...], dg)
        h = _bn_lrelu(h, s1_ref[...], s2_ref[...], g_ref[...], be_ref[...], N)
        an_out[...] = jnp.dot(h, wt_ref[...],
                              preferred_element_type=jnp.float32) + b_ref[...]
        bn_out[...] = jnp.dot(h, wb_ref[...],
                              preferred_element_type=jnp.float32)

    return pl.pallas_call(
        body,
        grid=(NB,),
        in_specs=[
            pl.BlockSpec((BR, C), lambda i: (i, 0)),
            pl.BlockSpec((BR, C), lambda i: (i, 0)),
            pl.BlockSpec((BR, C), lambda i: (i, 0)),
            pl.BlockSpec((BR, 16), lambda i: (i, 0)),
            pl.BlockSpec((8, H), lambda i: (0, 0)),
            pl.BlockSpec((8, H), lambda i: (0, 0)),
            pl.BlockSpec((1, H), lambda i: (0, 0)),
            pl.BlockSpec((1, H), lambda i: (0, 0)),
            pl.BlockSpec((H, CO), lambda i: (0, 0)),
            pl.BlockSpec((H, CO), lambda i: (0, 0)),
            pl.BlockSpec((1, CO), lambda i: (0, 0)),
        ],
        out_specs=[
            pl.BlockSpec((BR, CO), lambda i: (i, 0)),
            pl.BlockSpec((BR, CO), lambda i: (i, 0)),
        ],
        out_shape=[
            jax.ShapeDtypeStruct((NPAD, CO), jnp.float32),
            jax.ShapeDtypeStruct((NPAD, CO), jnp.float32),
        ],
    )(A, SB, MB, deg, s1, s2, g.reshape(1, H), be.reshape(1, H), wt, wb,
      b.reshape(1, CO))


def _tc_head_a(N, A, SB, MB, deg, s1, s2, g, be, wl1, bl1, wl2, bl2):
    NPAD, C = A.shape
    H = 3 * C
    H1 = wl1.shape[1]
    H2 = wl2.shape[1]
    BR, NB = _grid_rows(NPAD)

    def body(a_ref, sb_ref, mb_ref, dg_ref, s1_ref, s2_ref, g_ref, be_ref,
             w1_ref, b1_ref, w2_ref, b2_ref, a_out, s1_out, s2_out):
        i = pl.program_id(0)
        dg = dg_ref[...][:, 0:1]
        h = _combine(a_ref[...], sb_ref[...], mb_ref[...], dg)
        h = _bn_lrelu(h, s1_ref[...], s2_ref[...], g_ref[...], be_ref[...], N)
        t = jnp.dot(h, w1_ref[...],
                    preferred_element_type=jnp.float32) + b1_ref[...]
        t = jnp.where(t > 0.0, t, LRELU * t)
        a2 = jnp.dot(t, w2_ref[...],
                     preferred_element_type=jnp.float32) + b2_ref[...]
        a_out[...] = a2
        rows = lax.broadcasted_iota(jnp.int32, (BR, 1), 0) + i * BR
        am = jnp.where(rows < N, a2, 0.0)

        @pl.when(i == 0)
        def _():
            s1_out[...] = jnp.zeros_like(s1_out)
            s2_out[...] = jnp.zeros_like(s2_out)
        s1_out[0:1, :] += jnp.sum(am, axis=0, keepdims=True)
        s2_out[0:1, :] += jnp.sum(am * am, axis=0, keepdims=True)

    return pl.pallas_call(
        body,
        grid=(NB,),
        in_specs=[
            pl.BlockSpec((BR, C), lambda i: (i, 0)),
            pl.BlockSpec((BR, C), lambda i: (i, 0)),
            pl.BlockSpec((BR, C), lambda i: (i, 0)),
            pl.BlockSpec((BR, 16), lambda i: (i, 0)),
            pl.BlockSpec((8, H), lambda i: (0, 0)),
            pl.BlockSpec((8, H), lambda i: (0, 0)),
            pl.BlockSpec((1, H), lambda i: (0, 0)),
            pl.BlockSpec((1, H), lambda i: (0, 0)),
            pl.BlockSpec((H, H1), lambda i: (0, 0)),
            pl.BlockSpec((1, H1), lambda i: (0, 0)),
            pl.BlockSpec((H1, H2), lambda i: (0, 0)),
            pl.BlockSpec((1, H2), lambda i: (0, 0)),
        ],
        out_specs=[
            pl.BlockSpec((BR, H2), lambda i: (i, 0)),
            pl.BlockSpec((8, H2), lambda i: (0, 0)),
            pl.BlockSpec((8, H2), lambda i: (0, 0)),
        ],
        out_shape=[
            jax.ShapeDtypeStruct((NPAD, H2), jnp.float32),
            jax.ShapeDtypeStruct((8, H2), jnp.float32),
            jax.ShapeDtypeStruct((8, H2), jnp.float32),
        ],
    )(A, SB, MB, deg, s1, s2, g.reshape(1, H), be.reshape(1, H), wl1,
      bl1.reshape(1, H1), wl2, bl2.reshape(1, H2))


def _tc_head_b(N, a, s1, s2, g4, be4, wo, bo):
    NPAD, H2 = a.shape
    OUT = wo.shape[1]
    WPAD = 128
    BR, NB = _grid_rows(NPAD)
    wo_p = jnp.zeros((H2, WPAD), jnp.float32).at[:, :OUT].set(wo)
    bo_p = jnp.zeros((1, WPAD), jnp.float32).at[0, :OUT].set(bo)

    def body(a_ref, s1_ref, s2_ref, g_ref, be_ref, wo_ref, bo_ref, o_out):
        an = _bn_lrelu(a_ref[...], s1_ref[...], s2_ref[...], g_ref[...],
                       be_ref[...], N)
        o = jnp.dot(an, wo_ref[...],
                    preferred_element_type=jnp.float32) + bo_ref[...]
        cm = lax.broadcasted_iota(jnp.int32, (BR, WPAD), 1) < OUT
        mx = jnp.max(jnp.where(cm, o, -3e38), axis=1, keepdims=True)
        e = jnp.where(cm, jnp.exp(o - mx), 0.0)
        o_out[...] = e / jnp.sum(e, axis=1, keepdims=True)

    return pl.pallas_call(
        body,
        grid=(NB,),
        in_specs=[
            pl.BlockSpec((BR, H2), lambda i: (i, 0)),
            pl.BlockSpec((8, H2), lambda i: (0, 0)),
            pl.BlockSpec((8, H2), lambda i: (0, 0)),
            pl.BlockSpec((1, H2), lambda i: (0, 0)),
            pl.BlockSpec((1, H2), lambda i: (0, 0)),
            pl.BlockSpec((H2, WPAD), lambda i: (0, 0)),
            pl.BlockSpec((1, WPAD), lambda i: (0, 0)),
        ],
        out_specs=pl.BlockSpec((BR, WPAD), lambda i: (i, 0)),
        out_shape=jax.ShapeDtypeStruct((NPAD, WPAD), jnp.float32),
    )(a, s1, s2, g4.reshape(1, H2), be4.reshape(1, H2), wo_p, bo_p)


def kernel(x, edge_index, W1, b1, g1, be1, W2, b2, g2, be2, W3, b3, g3, be3,
           Wl1, bl1, Wl2, bl2, g4, be4, Wo, bo):
    N, F = x.shape
    E = edge_index.shape[1]
    R = -(-N // (NW * 16)) * 16
    NPAD = NW * R
    EB = 6400 if E % 6400 == 0 else E
    C = W1.shape[1]

    xp = jnp.pad(x, ((0, NPAD - N), (0, 0)))

    srcl, dstl, cnts, deg = _make_prep(E, R, EB)(
        edge_index[0].reshape(E), edge_index[1].reshape(E))
    scat = _make_scatter(R, C)

    A, B = _tc_pre(xp, W1[:F], W1[F:], b1)
    SB, MB = scat(B, srcl, dstl, cnts)
    s1, s2 = _tc_stats(N, A, SB, MB, deg)
    A, B = _tc_apply(N, A, SB, MB, deg, s1, s2, g1, be1,
                     W2[:3 * C], W2[3 * C:], b2)

    SB, MB = scat(B, srcl, dstl, cnts)
    s1, s2 = _tc_stats(N, A, SB, MB, deg)
    A, B = _tc_apply(N, A, SB, MB, deg, s1, s2, g2, be2,
                     W3[:3 * C], W3[3 * C:], b3)

    SB, MB = scat(B, srcl, dstl, cnts)
    s1, s2 = _tc_stats(N, A, SB, MB, deg)
    a, h1, h2 = _tc_head_a(N, A, SB, MB, deg, s1, s2, g3, be3,
                           Wl1, bl1, Wl2, bl2)
    out = _tc_head_b(N, a, h1, h2, g4, be4, Wo, bo)
    return out[:N, :Wo.shape[1]]
